# baseline (speedup 1.0000x reference)
import jax
import jax.numpy as jnp
from jax.experimental import pallas as pl
from jax.experimental.pallas import tpu as pltpu

N = 1024
HID = 128
V = 2
F = HID // 2
BLOCK_D = 256
GRID = N // BLOCK_D
SPLITS = 2

_NORMAL = (((1,), (0,)), ((), ()))
_T_DIMNUMS = (((0,), (0,)), ((), ()))


def _bignn_kernel(x_ref, afw_ref, abw_ref, wfw_ref, bfw_ref, wbw_ref,
                  bbw_ref, w1_ref, b1_ref, out_ref,
                  hfw_ref, hbw_ref, sum_ref, buf_fw, buf_bw,
                  sem_fw, sem_bw):
    def _copies(j):
        d = pl.ds(j * BLOCK_D, BLOCK_D)
        H = N // SPLITS
        cs = []
        for src, buf, sem in ((afw_ref, buf_fw, sem_fw),
                              (abw_ref, buf_bw, sem_bw)):
            for i in range(V):
                for s in range(SPLITS):
                    r = pl.ds(s * H, H)
                    cs.append(pltpu.make_async_copy(
                        src.at[i, r, d], buf.at[j, i, r],
                        sem.at[j, i, s]))
        return cs

    for j in range(GRID):
        for c in _copies(j):
            c.start()

    x = x_ref[...]
    for w_ref, b_ref, h_ref in ((wfw_ref, bfw_ref, hfw_ref),
                                (wbw_ref, bbw_ref, hbw_ref)):
        for i in range(V):
            h = (jnp.dot(x, w_ref[i], preferred_element_type=jnp.float32)
                 + b_ref[i:i + 1, :])
            h_ref[:, pl.ds(i * N, N)] = jnp.swapaxes(h, 0, 1)

    for j in range(GRID):
        for c in _copies(j):
            c.wait()

        parts = []
        for buf, h_ref in ((buf_bw, hbw_ref), (buf_fw, hfw_ref)):
            acc = None
            for i in range(V):
                a = buf[j, i].astype(jnp.float32)
                agg_t = jax.lax.dot_general(
                    h_ref[:, pl.ds(i * N, N)], a, _NORMAL,
                    precision=jax.lax.Precision.DEFAULT,
                    preferred_element_type=jnp.float32)
                r = jnp.maximum(agg_t, 0.0)
                acc = r if acc is None else acc + r
            parts.append(acc)
        sum_ref[:, pl.ds(j * BLOCK_D, BLOCK_D)] = jnp.concatenate(
            parts, axis=0)

    out_ref[...] = (
        jax.lax.dot_general(sum_ref[...], w1_ref[...], _T_DIMNUMS,
                            preferred_element_type=jnp.float32)
        + b1_ref[...] + x)


@jax.jit
def kernel(inps, fw_adjs, bw_adjs, W_fw, b_fw, W_bw, b_bw, W1, b1):
    out = pl.pallas_call(
        _bignn_kernel,
        in_specs=[
            pl.BlockSpec(memory_space=pltpu.MemorySpace.VMEM),
            pl.BlockSpec(memory_space=pl.ANY),
            pl.BlockSpec(memory_space=pl.ANY),
            pl.BlockSpec(memory_space=pltpu.MemorySpace.VMEM),
            pl.BlockSpec(memory_space=pltpu.MemorySpace.VMEM),
            pl.BlockSpec(memory_space=pltpu.MemorySpace.VMEM),
            pl.BlockSpec(memory_space=pltpu.MemorySpace.VMEM),
            pl.BlockSpec(memory_space=pltpu.MemorySpace.VMEM),
            pl.BlockSpec(memory_space=pltpu.MemorySpace.VMEM),
        ],
        out_specs=pl.BlockSpec(memory_space=pltpu.MemorySpace.VMEM),
        out_shape=jax.ShapeDtypeStruct((N, HID), jnp.float32),
        scratch_shapes=[
            pltpu.VMEM((F, V * N), jnp.float32),
            pltpu.VMEM((F, V * N), jnp.float32),
            pltpu.VMEM((HID, N), jnp.float32),
            pltpu.VMEM((GRID, V, N, BLOCK_D), jnp.int32),
            pltpu.VMEM((GRID, V, N, BLOCK_D), jnp.int32),
            pltpu.SemaphoreType.DMA((GRID, V, SPLITS)),
            pltpu.SemaphoreType.DMA((GRID, V, SPLITS)),
        ],
    )(inps, fw_adjs, bw_adjs, W_fw, b_fw, W_bw, b_bw, W1,
      b1.reshape(1, HID))
    return out
